# Initial kernel scaffold; baseline (speedup 1.0000x reference)
import jax
import jax.numpy as jnp
from jax.experimental import pallas as pl


def kernel(x, thresholds, lut_w1, lut_w2, idx1, idx2):
    raise NotImplementedError("write your pallas kernel here")


# speedup vs baseline: 1.2887x; 1.2887x over previous
import functools
import math

import jax
import jax.numpy as jnp
from jax import lax
from jax.experimental import pallas as pl
from jax.experimental.pallas import tpu as pltpu
from jax.experimental.pallas import tpu_sc as plsc

_B = 1024
_F = 784
_T = 3
_H1 = 2000
_H2 = 1000
_N = 6
_K = 10
_TAU = 1.0 / 0.3
_LANES = 16
_NW = 32
_ROWS = _B // _NW
_BV = _ROWS // _LANES
_C1 = 40
_C2 = 20
_G = _H2 // _K


def _body(x_hbm, thr_hbm, w1_hbm, w2_hbm, idx1_hbm, idx2_hbm, out_hbm,
          x_v, h1_v, idx1_v, thr_v, idx2_v, sig1c_v, sig2c_v, stage_v,
          out_v, sp1, sp2):
    cid = lax.axis_index("c")
    sid = lax.axis_index("s")
    wid = cid * 16 + sid
    iota = lax.iota(jnp.int32, _LANES)

    pltpu.sync_copy(idx1_hbm, idx1_v)
    pltpu.sync_copy(thr_hbm, thr_v)
    pltpu.sync_copy(idx2_hbm, idx2_v)
    pltpu.sync_copy(x_hbm.at[pl.ds(wid * _ROWS * _F, _ROWS * _F)], x_v)

    def _stage(w_hbm, sp, num_chunks):
        def chunk(i, _):
            c = i * 16 + sid

            @pl.when(c < num_chunks)
            def _():
                start = c * 512
                pltpu.sync_copy(w_hbm.at[pl.ds(start, 512)], stage_v)

                def svec(k, _):
                    v = stage_v[pl.ds(k * _LANES, _LANES)]
                    stage_v[pl.ds(k * _LANES, _LANES)] = (
                        1.0 / (1.0 + jnp.exp(-v)))
                    return 0

                lax.fori_loop(0, 512 // _LANES, svec, 0)
                pltpu.sync_copy(stage_v, sp.at[pl.ds(start, 512)])

            return 0

        lax.fori_loop(0, (num_chunks + 15) // 16, chunk, 0)

    _stage(w1_hbm, sp1, _H1 * 64 // 512)
    _stage(w2_hbm, sp2, _H2 * 64 // 512)

    plsc.subcore_barrier()

    rowbase = [(iota + bv * _LANES) * _F for bv in range(_BV)]
    colbase = [iota + bv * _LANES for bv in range(_BV)]

    def l1chunk(ch, _):
        pltpu.sync_copy(sp1.at[pl.ds(ch * _C1 * 64, _C1 * 64)], sig1c_v)

        def l1body(o, _):
            og = ch * _C1 + o
            codes = [jnp.zeros((_LANES,), jnp.int32) for _ in range(_BV)]
            for j in range(_N):
                av = plsc.load_gather(
                    idx1_v, [jnp.full((_LANES,), og * _N + j, jnp.int32)])
                fvec = av // _T
                rvec = av - fvec * _T
                thv = plsc.load_gather(thr_v, [fvec * _T + rvec])
                for bv in range(_BV):
                    xv = plsc.load_gather(x_v, [rowbase[bv] + fvec])
                    codes[bv] = codes[bv] + jnp.where(xv > thv, 1 << j, 0)
            obase = jnp.full((_LANES,), o * 64, jnp.int32)
            for bv in range(_BV):
                h = plsc.load_gather(sig1c_v, [obase + codes[bv]])
                h1_v[pl.ds(og * _ROWS + bv * _LANES, _LANES)] = h
            return 0

        lax.fori_loop(0, _C1, l1body, 0)
        return 0

    lax.fori_loop(0, _H1 // _C1, l1chunk, 0)

    def _run_group(g, _):
        def l2sub(sub, accs):
            pltpu.sync_copy(
                sp2.at[pl.ds((g * _G + sub * _C2) * 64, _C2 * 64)], sig2c_v)

            def l2body(o2, accs):
                return _l2body(g * _G + sub * _C2 + o2, o2, accs)

            return lax.fori_loop(0, _C2, l2body, accs)

        def _l2body(o2g, o2, accs):
            svecs = [[] for _ in range(_BV)]
            for j in range(_N):
                av = plsc.load_gather(
                    idx2_v, [jnp.full((_LANES,), o2g * _N + j, jnp.int32)])
                abase = av * _ROWS
                for bv in range(_BV):
                    svecs[bv].append(
                        plsc.load_gather(h1_v, [abase + colbase[bv]]))
            tbase = o2 * 64
            los = []
            dels = []
            for c in range(32):
                lo = plsc.load_gather(
                    sig2c_v,
                    [jnp.full((_LANES,), tbase + 2 * c, jnp.int32)])
                hi = plsc.load_gather(
                    sig2c_v,
                    [jnp.full((_LANES,), tbase + 2 * c + 1, jnp.int32)])
                los.append(lo)
                dels.append(hi - lo)

            out = []
            for bv in range(_BV):
                s = svecs[bv]

                def rec(base, size):
                    if size == 2:
                        c = base // 2
                        return los[c] + s[0] * dels[c]
                    half = size // 2
                    j = int(math.log2(size)) - 1
                    lo = rec(base, half)
                    hi = rec(base + half, half)
                    return lo + s[j] * (hi - lo)

                out.append(accs[bv] + rec(0, 64))
            return tuple(out)

        accs = lax.fori_loop(
            0, _G // _C2, l2sub,
            tuple(jnp.zeros((_LANES,), jnp.float32) for _ in range(_BV)))
        inv_tau = jnp.float32(1.0 / _TAU)
        for bv in range(_BV):
            plsc.store_scatter(
                out_v, [(iota + bv * _LANES) * _K + g], accs[bv] * inv_tau)
        return 0

    lax.fori_loop(0, _K, _run_group, 0)

    pltpu.sync_copy(out_v, out_hbm.at[pl.ds(wid * _ROWS * _K, _ROWS * _K)])


_mesh = plsc.VectorSubcoreMesh(core_axis_name="c", subcore_axis_name="s")

_dwn = functools.partial(
    pl.kernel,
    out_type=jax.ShapeDtypeStruct((_B * _K,), jnp.float32),
    mesh=_mesh,
    compiler_params=pltpu.CompilerParams(needs_layout_passes=False),
    scratch_types=[
        pltpu.VMEM((_ROWS * _F,), jnp.float32),
        pltpu.VMEM((_H1 * _ROWS,), jnp.float32),
        pltpu.VMEM((_H1 * _N,), jnp.int32),
        pltpu.VMEM((_F * _T,), jnp.float32),
        pltpu.VMEM((_H2 * _N,), jnp.int32),
        pltpu.VMEM((_C1 * 64,), jnp.float32),
        pltpu.VMEM((_C2 * 64,), jnp.float32),
        pltpu.VMEM((512,), jnp.float32),
        pltpu.VMEM((_ROWS * _K,), jnp.float32),
        pltpu.VMEM_SHARED((_H1 * 64,), jnp.float32),
        pltpu.VMEM_SHARED((_H2 * 64,), jnp.float32),
    ],
)(_body)


def kernel(x, thresholds, lut_w1, lut_w2, idx1, idx2):
    out = _dwn(x.reshape(-1), thresholds.reshape(-1), lut_w1.reshape(-1),
               lut_w2.reshape(-1), idx1.reshape(-1), idx2.reshape(-1))
    return out.reshape(_B, _K)


# speedup vs baseline: 2.6665x; 2.0692x over previous
import functools
import math

import jax
import jax.numpy as jnp
from jax import lax
from jax.experimental import pallas as pl
from jax.experimental.pallas import tpu as pltpu
from jax.experimental.pallas import tpu_sc as plsc

_B = 1024
_F = 784
_T = 3
_H1 = 2000
_H2 = 1000
_N = 6
_K = 10
_TAU = 1.0 / 0.3
_LANES = 16
_NW = 32
_ROWS = _B // _NW
_BV = _ROWS // _LANES
_C1 = 40
_C2 = 20
_G = _H2 // _K


def _body(x_hbm, thr_hbm, w1_hbm, w2_hbm, idx1_hbm, idx2_hbm, out_hbm,
          x_v, h1_v, idx1_v, thr_v, idx2_v, sig1c_v, sig2c_v, stage_v,
          out_v, sp1, sp2):
    cid = lax.axis_index("c")
    sid = lax.axis_index("s")
    wid = cid * 16 + sid
    iota = lax.iota(jnp.int32, _LANES)

    pltpu.sync_copy(idx1_hbm, idx1_v)
    pltpu.sync_copy(thr_hbm, thr_v)
    pltpu.sync_copy(idx2_hbm, idx2_v)
    pltpu.sync_copy(x_hbm.at[pl.ds(wid * _ROWS * _F, _ROWS * _F)], x_v)

    def _stage(w_hbm, sp, num_chunks):
        def chunk(i, _):
            c = i * 16 + sid

            @pl.when(c < num_chunks)
            def _():
                start = c * 512
                pltpu.sync_copy(w_hbm.at[pl.ds(start, 512)], stage_v)

                def svec(k, _):
                    v = stage_v[pl.ds(k * _LANES, _LANES)]
                    stage_v[pl.ds(k * _LANES, _LANES)] = (
                        1.0 / (1.0 + jnp.exp(-v)))
                    return 0

                lax.fori_loop(0, 512 // _LANES, svec, 0)
                pltpu.sync_copy(stage_v, sp.at[pl.ds(start, 512)])

            return 0

        lax.fori_loop(0, (num_chunks + 15) // 16, chunk, 0)

    _stage(w1_hbm, sp1, _H1 * 64 // 512)
    _stage(w2_hbm, sp2, _H2 * 64 // 512)

    plsc.subcore_barrier()

    rowbase = [(iota + bv * _LANES) * _F for bv in range(_BV)]
    colbase = [iota + bv * _LANES for bv in range(_BV)]

    def l1chunk(ch, _):
        pltpu.sync_copy(sp1.at[pl.ds(ch * _C1 * 64, _C1 * 64)], sig1c_v)

        def l1body(o, _):
            og = ch * _C1 + o
            codes = [jnp.zeros((_LANES,), jnp.int32) for _ in range(_BV)]
            for j in range(_N):
                av = plsc.load_gather(
                    idx1_v, [jnp.full((_LANES,), og * _N + j, jnp.int32)])
                fvec = lax.shift_right_logical(av * 21846, 16)
                thv = plsc.load_gather(thr_v, [av])
                for bv in range(_BV):
                    xv = plsc.load_gather(x_v, [rowbase[bv] + fvec])
                    codes[bv] = codes[bv] + jnp.where(xv > thv, 1 << j, 0)
            obase = jnp.full((_LANES,), o * 64, jnp.int32)
            for bv in range(_BV):
                h = plsc.load_gather(sig1c_v, [obase + codes[bv]])
                h1_v[pl.ds(og * _ROWS + bv * _LANES, _LANES)] = h
            return 0

        lax.fori_loop(0, _C1, l1body, 0)
        return 0

    lax.fori_loop(0, _H1 // _C1, l1chunk, 0)

    def _run_group(g, _):
        def l2sub(sub, accs):
            pltpu.sync_copy(
                sp2.at[pl.ds((g * _G + sub * _C2) * 64, _C2 * 64)], sig2c_v)

            def l2body(o2, accs):
                return _l2body(g * _G + sub * _C2 + o2, o2, accs)

            return lax.fori_loop(0, _C2, l2body, accs)

        def _l2body(o2g, o2, accs):
            svecs = [[] for _ in range(_BV)]
            for j in range(_N):
                av = plsc.load_gather(
                    idx2_v, [jnp.full((_LANES,), o2g * _N + j, jnp.int32)])
                abase = av * _ROWS
                for bv in range(_BV):
                    svecs[bv].append(
                        plsc.load_gather(h1_v, [abase + colbase[bv]]))
            tbase = o2 * 64
            los = []
            dels = []
            for c in range(32):
                lo = plsc.load_gather(
                    sig2c_v,
                    [jnp.full((_LANES,), tbase + 2 * c, jnp.int32)])
                hi = plsc.load_gather(
                    sig2c_v,
                    [jnp.full((_LANES,), tbase + 2 * c + 1, jnp.int32)])
                los.append(lo)
                dels.append(hi - lo)

            out = []
            for bv in range(_BV):
                s = svecs[bv]

                def rec(base, size):
                    if size == 2:
                        c = base // 2
                        return los[c] + s[0] * dels[c]
                    half = size // 2
                    j = int(math.log2(size)) - 1
                    lo = rec(base, half)
                    hi = rec(base + half, half)
                    return lo + s[j] * (hi - lo)

                out.append(accs[bv] + rec(0, 64))
            return tuple(out)

        accs = lax.fori_loop(
            0, _G // _C2, l2sub,
            tuple(jnp.zeros((_LANES,), jnp.float32) for _ in range(_BV)))
        inv_tau = jnp.float32(1.0 / _TAU)
        for bv in range(_BV):
            plsc.store_scatter(
                out_v, [(iota + bv * _LANES) * _K + g], accs[bv] * inv_tau)
        return 0

    lax.fori_loop(0, _K, _run_group, 0)

    pltpu.sync_copy(out_v, out_hbm.at[pl.ds(wid * _ROWS * _K, _ROWS * _K)])


_mesh = plsc.VectorSubcoreMesh(core_axis_name="c", subcore_axis_name="s")

_dwn = functools.partial(
    pl.kernel,
    out_type=jax.ShapeDtypeStruct((_B * _K,), jnp.float32),
    mesh=_mesh,
    compiler_params=pltpu.CompilerParams(needs_layout_passes=False),
    scratch_types=[
        pltpu.VMEM((_ROWS * _F,), jnp.float32),
        pltpu.VMEM((_H1 * _ROWS,), jnp.float32),
        pltpu.VMEM((_H1 * _N,), jnp.int32),
        pltpu.VMEM((_F * _T,), jnp.float32),
        pltpu.VMEM((_H2 * _N,), jnp.int32),
        pltpu.VMEM((_C1 * 64,), jnp.float32),
        pltpu.VMEM((_C2 * 64,), jnp.float32),
        pltpu.VMEM((512,), jnp.float32),
        pltpu.VMEM((_ROWS * _K,), jnp.float32),
        pltpu.VMEM_SHARED((_H1 * 64,), jnp.float32),
        pltpu.VMEM_SHARED((_H2 * 64,), jnp.float32),
    ],
)(_body)


def kernel(x, thresholds, lut_w1, lut_w2, idx1, idx2):
    out = _dwn(x.reshape(-1), thresholds.reshape(-1), lut_w1.reshape(-1),
               lut_w2.reshape(-1), idx1.reshape(-1), idx2.reshape(-1))
    return out.reshape(_B, _K)
